# baseline (speedup 1.0000x reference)
import functools

import jax
import jax.numpy as jnp
from jax import lax
from jax.experimental import pallas as pl
from jax.experimental.pallas import tpu as pltpu
from jax.experimental.pallas import tpu_sc as plsc

_K = 64
_GAMMA = 1.0
_N = 10000
_D = 256
_A = 1024
_TN = 2048
_NPAD = 10240
_TA = 128
_NI = _A // _TA
_NJ = _NPAD // _TN


def _sc_gather(t1, t2, i1, i2):
    info = plsc.get_sparse_core_info()
    nc, ns = info.num_cores, info.num_subcores
    nw = nc * ns
    bpw = _A // nw
    mesh = plsc.VectorSubcoreMesh(core_axis_name="c", subcore_axis_name="s")

    @functools.partial(
        pl.kernel,
        mesh=mesh,
        out_type=[
            jax.ShapeDtypeStruct((_A, _D), jnp.float32),
            jax.ShapeDtypeStruct((_A, _D), jnp.float32),
        ],
        scratch_types=[
            pltpu.VMEM((bpw,), jnp.int32),
            pltpu.VMEM((bpw, _D), jnp.float32),
            pltpu.SemaphoreType.DMA,
        ],
    )
    def gk(t1_hbm, t2_hbm, i1_hbm, i2_hbm, o1_hbm, o2_hbm, idx_v, rows_v, sem):
        wid = lax.axis_index("s") * nc + lax.axis_index("c")
        base = wid * bpw
        pltpu.sync_copy(i1_hbm.at[pl.ds(base, bpw)], idx_v)
        pltpu.async_copy(t1_hbm.at[idx_v], rows_v, sem).wait()
        pltpu.sync_copy(rows_v, o1_hbm.at[pl.ds(base, bpw)])
        pltpu.sync_copy(i2_hbm.at[pl.ds(base, bpw)], idx_v)
        pltpu.async_copy(t2_hbm.at[idx_v], rows_v, sem).wait()
        pltpu.sync_copy(rows_v, o2_hbm.at[pl.ds(base, bpw)])

    return gk(t1, t2, i1, i2)


_NSC = 256


def _tc_body(ct_ref, aet_ref, ae1_ref, ae2_ref, out_ref, dist_ref, acs_ref):
    j = pl.program_id(2)

    def grp_body(grp, carry):
        at = aet_ref[0]
        for kk in range(8):
            row_iota = lax.broadcasted_iota(jnp.int32, (_TA, 1), 0)
            oh = (row_iota == grp * 8 + kk).astype(jnp.float32)
            acs_ref[kk] = lax.dot_general(
                at, oh, (((1,), (0,)), ((), ())),
                preferred_element_type=jnp.float32,
            )

        def nc_body(nc, carry2):
            nbase = j * _TN + nc * _NSC
            accs = [jnp.zeros((8, _NSC), jnp.float32) for _ in range(8)]
            for dt in range(_D // 8):
                ct_chunk = ct_ref[0, dt * 8:(dt + 1) * 8,
                                  pl.ds(nc * _NSC, _NSC)]
                for kk in range(8):
                    ac = acs_ref[kk, dt * 8:(dt + 1) * 8, :]
                    accs[kk] = accs[kk] + jnp.abs(ct_chunk - ac)
            rows = [jnp.sum(a, axis=0, keepdims=True) for a in accs]
            blk = jnp.concatenate(rows, axis=0)
            dist_ref[pl.ds(pl.multiple_of(grp * 8, 8), 8),
                     pl.ds(nbase, _NSC)] = blk
            return carry2

        lax.fori_loop(0, _TN // _NSC, nc_body, 0)
        return carry

    lax.fori_loop(0, _TA // 8, grp_body, 0)

    @pl.when(j == _NJ - 1)
    def _select_and_reduce():
        dvec = _GAMMA + jnp.sum(
            jnp.abs(ae1_ref[...] - ae2_ref[...]), axis=1, keepdims=True
        )

        def _di():
            return lax.bitcast_convert_type(dist_ref[...], jnp.int32)

        lo0 = jnp.min(_di(), axis=1, keepdims=True)
        hi0 = jnp.max(_di(), axis=1, keepdims=True)

        def bs_cond(lo_hi):
            lo, hi = lo_hi
            return jnp.any(lo < hi)

        def bs_body(lo_hi):
            lo, hi = lo_hi
            mid = lo + lax.shift_right_arithmetic(hi - lo, 1)
            cnt = jnp.sum((_di() <= mid).astype(jnp.int32),
                          axis=1, keepdims=True)
            ge = cnt >= _K
            return jnp.where(ge, lo, mid + 1), jnp.where(ge, mid, hi)

        tau_i, _ = lax.while_loop(bs_cond, bs_body, (lo0, hi0))
        tau_f = lax.bitcast_convert_type(tau_i, jnp.float32)

        d_f = dist_ref[...]
        below = _di() < tau_i
        c = jnp.sum(below.astype(jnp.int32), axis=1, keepdims=True)
        contrib = jnp.where(below, jnp.maximum(dvec - d_f, 0.0), 0.0)
        s = jnp.sum(contrib, axis=1, keepdims=True)
        total = s + (_K - c).astype(jnp.float32) * jnp.maximum(dvec - tau_f, 0.0)
        out_ref[...] = total


def _run_tc(ct, aet, ae1, ae2, interpret=False):
    return pl.pallas_call(
        _tc_body,
        grid=(2, _NI, _NJ),
        in_specs=[
            pl.BlockSpec((1, _D, _TN), lambda g, i, j: (g, 0, j)),
            pl.BlockSpec((1, _D, _TA), lambda g, i, j: (g, 0, i)),
            pl.BlockSpec((_TA, _D), lambda g, i, j: (i, 0)),
            pl.BlockSpec((_TA, _D), lambda g, i, j: (i, 0)),
        ],
        out_specs=pl.BlockSpec((_TA, 1), lambda g, i, j: (g * _NI + i, 0)),
        out_shape=jax.ShapeDtypeStruct((2 * _A, 1), jnp.float32),
        scratch_shapes=[pltpu.VMEM((_TA, _NPAD), jnp.float32),
                        pltpu.VMEM((8, _D, 1), jnp.float32)],
        interpret=interpret,
    )(ct, aet, ae1, ae2)


def kernel(out1, out2, anchor1, anchor2):
    ae1, ae2 = _sc_gather(
        out1, out2, anchor1.astype(jnp.int32), anchor2.astype(jnp.int32)
    )
    pad = jnp.full((_NPAD - _N, _D), jnp.inf, jnp.float32)
    c2 = jnp.concatenate([out2, pad], axis=0)
    c1 = jnp.concatenate([out1, pad], axis=0)
    ct = jnp.stack([c2.T, c1.T])
    aet = jnp.stack([ae1.T, ae2.T])
    partial = _run_tc(ct, aet, ae1, ae2)
    return jnp.sum(partial) / (_A * _K)


# speedup vs baseline: 1.2598x; 1.2598x over previous
import functools

import jax
import jax.numpy as jnp
from jax import lax
from jax.experimental import pallas as pl
from jax.experimental.pallas import tpu as pltpu
from jax.experimental.pallas import tpu_sc as plsc

_K = 64
_GAMMA = 1.0
_N = 10000
_D = 256
_A = 1024
_TN = 2048
_NPAD = 10240
_TA = 128
_NI = _A // _TA
_NJ = _NPAD // _TN


def _sc_gather(t1, t2, i1, i2):
    info = plsc.get_sparse_core_info()
    nc, ns = info.num_cores, info.num_subcores
    nw = nc * ns
    bpw = _A // nw
    mesh = plsc.VectorSubcoreMesh(core_axis_name="c", subcore_axis_name="s")

    @functools.partial(
        pl.kernel,
        mesh=mesh,
        out_type=[
            jax.ShapeDtypeStruct((_A, _D), jnp.float32),
            jax.ShapeDtypeStruct((_A, _D), jnp.float32),
        ],
        scratch_types=[
            pltpu.VMEM((bpw,), jnp.int32),
            pltpu.VMEM((bpw, _D), jnp.float32),
            pltpu.SemaphoreType.DMA,
        ],
    )
    def gk(t1_hbm, t2_hbm, i1_hbm, i2_hbm, o1_hbm, o2_hbm, idx_v, rows_v, sem):
        wid = lax.axis_index("s") * nc + lax.axis_index("c")
        base = wid * bpw
        pltpu.sync_copy(i1_hbm.at[pl.ds(base, bpw)], idx_v)
        pltpu.async_copy(t1_hbm.at[idx_v], rows_v, sem).wait()
        pltpu.sync_copy(rows_v, o1_hbm.at[pl.ds(base, bpw)])
        pltpu.sync_copy(i2_hbm.at[pl.ds(base, bpw)], idx_v)
        pltpu.async_copy(t2_hbm.at[idx_v], rows_v, sem).wait()
        pltpu.sync_copy(rows_v, o2_hbm.at[pl.ds(base, bpw)])

    return gk(t1, t2, i1, i2)


_NSC = 512


def _tc_body(ct_ref, aet_ref, ae1_ref, ae2_ref, out_ref, dist_ref, acs_ref):
    j = pl.program_id(2)

    def grp_body(grp, carry):
        at = aet_ref[0]
        row_iota = lax.broadcasted_iota(jnp.int32, (_TA, 8), 0)
        col_iota = lax.broadcasted_iota(jnp.int32, (_TA, 8), 1)
        oh = (row_iota == grp * 8 + col_iota).astype(jnp.float32)
        acs_ref[...] = lax.dot_general(
            at, oh, (((1,), (0,)), ((), ())),
            preferred_element_type=jnp.float32,
        )

        def nc_body(nc, carry2):
            nbase = j * _TN + nc * _NSC
            accs = [jnp.zeros((8, _NSC), jnp.float32) for _ in range(8)]
            for dt in range(_D // 8):
                ct_chunk = ct_ref[0, dt * 8:(dt + 1) * 8,
                                  pl.ds(nc * _NSC, _NSC)]
                for kk in range(8):
                    ac = acs_ref[dt * 8:(dt + 1) * 8, kk:kk + 1]
                    accs[kk] = accs[kk] + jnp.abs(ct_chunk - ac)
            rows = [jnp.sum(a, axis=0, keepdims=True) for a in accs]
            blk = jnp.concatenate(rows, axis=0)
            dist_ref[pl.ds(pl.multiple_of(grp * 8, 8), 8),
                     pl.ds(nbase, _NSC)] = blk
            return carry2

        lax.fori_loop(0, _TN // _NSC, nc_body, 0)
        return carry

    lax.fori_loop(0, _TA // 8, grp_body, 0)

    @pl.when(j == _NJ - 1)
    def _select_and_reduce():
        dvec = _GAMMA + jnp.sum(
            jnp.abs(ae1_ref[...] - ae2_ref[...]), axis=1, keepdims=True
        )

        def _di():
            return lax.bitcast_convert_type(dist_ref[...], jnp.int32)

        lo0 = jnp.min(_di(), axis=1, keepdims=True)
        hi0 = jnp.max(_di(), axis=1, keepdims=True)

        def bs_cond(lo_hi):
            lo, hi = lo_hi
            return jnp.any(lo < hi)

        def bs_body(lo_hi):
            lo, hi = lo_hi
            mid = lo + lax.shift_right_arithmetic(hi - lo, 1)
            cnt = jnp.sum((_di() <= mid).astype(jnp.int32),
                          axis=1, keepdims=True)
            ge = cnt >= _K
            return jnp.where(ge, lo, mid + 1), jnp.where(ge, mid, hi)

        tau_i, _ = lax.while_loop(bs_cond, bs_body, (lo0, hi0))
        tau_f = lax.bitcast_convert_type(tau_i, jnp.float32)

        d_f = dist_ref[...]
        below = _di() < tau_i
        c = jnp.sum(below.astype(jnp.int32), axis=1, keepdims=True)
        contrib = jnp.where(below, jnp.maximum(dvec - d_f, 0.0), 0.0)
        s = jnp.sum(contrib, axis=1, keepdims=True)
        total = s + (_K - c).astype(jnp.float32) * jnp.maximum(dvec - tau_f, 0.0)
        out_ref[...] = total


def _run_tc(ct, aet, ae1, ae2, interpret=False):
    return pl.pallas_call(
        _tc_body,
        grid=(2, _NI, _NJ),
        in_specs=[
            pl.BlockSpec((1, _D, _TN), lambda g, i, j: (g, 0, j)),
            pl.BlockSpec((1, _D, _TA), lambda g, i, j: (g, 0, i)),
            pl.BlockSpec((_TA, _D), lambda g, i, j: (i, 0)),
            pl.BlockSpec((_TA, _D), lambda g, i, j: (i, 0)),
        ],
        out_specs=pl.BlockSpec((_TA, 1), lambda g, i, j: (g * _NI + i, 0)),
        out_shape=jax.ShapeDtypeStruct((2 * _A, 1), jnp.float32),
        scratch_shapes=[pltpu.VMEM((_TA, _NPAD), jnp.float32),
                        pltpu.VMEM((_D, 8), jnp.float32)],
        interpret=interpret,
    )(ct, aet, ae1, ae2)


def kernel(out1, out2, anchor1, anchor2):
    ae1, ae2 = _sc_gather(
        out1, out2, anchor1.astype(jnp.int32), anchor2.astype(jnp.int32)
    )
    pad = jnp.full((_NPAD - _N, _D), jnp.inf, jnp.float32)
    c2 = jnp.concatenate([out2, pad], axis=0)
    c1 = jnp.concatenate([out1, pad], axis=0)
    ct = jnp.stack([c2.T, c1.T])
    aet = jnp.stack([ae1.T, ae2.T])
    partial = _run_tc(ct, aet, ae1, ae2)
    return jnp.sum(partial) / (_A * _K)


# speedup vs baseline: 1.2825x; 1.0181x over previous
import functools

import jax
import jax.numpy as jnp
from jax import lax
from jax.experimental import pallas as pl
from jax.experimental.pallas import tpu as pltpu
from jax.experimental.pallas import tpu_sc as plsc

_K = 64
_GAMMA = 1.0
_N = 10000
_D = 256
_A = 1024
_TN = 2048
_NPAD = 10240
_TA = 128
_NI = _A // _TA
_NJ = _NPAD // _TN


def _sc_gather(t1, t2, i1, i2):
    info = plsc.get_sparse_core_info()
    nc, ns = info.num_cores, info.num_subcores
    nw = nc * ns
    bpw = _A // nw
    mesh = plsc.VectorSubcoreMesh(core_axis_name="c", subcore_axis_name="s")

    @functools.partial(
        pl.kernel,
        mesh=mesh,
        out_type=[
            jax.ShapeDtypeStruct((_A, _D), jnp.float32),
            jax.ShapeDtypeStruct((_A, _D), jnp.float32),
        ],
        scratch_types=[
            pltpu.VMEM((bpw,), jnp.int32),
            pltpu.VMEM((bpw, _D), jnp.float32),
            pltpu.SemaphoreType.DMA,
        ],
    )
    def gk(t1_hbm, t2_hbm, i1_hbm, i2_hbm, o1_hbm, o2_hbm, idx_v, rows_v, sem):
        wid = lax.axis_index("s") * nc + lax.axis_index("c")
        base = wid * bpw
        pltpu.sync_copy(i1_hbm.at[pl.ds(base, bpw)], idx_v)
        pltpu.async_copy(t1_hbm.at[idx_v], rows_v, sem).wait()
        pltpu.sync_copy(rows_v, o1_hbm.at[pl.ds(base, bpw)])
        pltpu.sync_copy(i2_hbm.at[pl.ds(base, bpw)], idx_v)
        pltpu.async_copy(t2_hbm.at[idx_v], rows_v, sem).wait()
        pltpu.sync_copy(rows_v, o2_hbm.at[pl.ds(base, bpw)])

    return gk(t1, t2, i1, i2)


_NSC = 512


def _tc_body(ct_ref, aet_ref, ae1_ref, ae2_ref, out_ref, dist_ref, acs_ref, rowbuf_ref):
    j = pl.program_id(2)

    def grp_body(grp, carry):
        at = aet_ref[0]
        for kk in range(8):
            row_iota = lax.broadcasted_iota(jnp.int32, (_TA, 1), 0)
            oh = (row_iota == grp * 8 + kk).astype(jnp.float32)
            acs_ref[kk] = lax.dot_general(
                at, oh, (((1,), (0,)), ((), ())),
                preferred_element_type=jnp.float32,
            )

        def nc_body(nc, carry2):
            nbase = j * _TN + nc * _NSC
            for sg in range(2):
                accs = [jnp.zeros((8, _NSC), jnp.float32) for _ in range(4)]
                for dt in range(_D // 8):
                    ct_chunk = ct_ref[0, dt * 8:(dt + 1) * 8,
                                      pl.ds(nc * _NSC, _NSC)]
                    for k4 in range(4):
                        kk = sg * 4 + k4
                        ac = acs_ref[kk, dt * 8:(dt + 1) * 8, :]
                        accs[k4] = accs[k4] + jnp.abs(ct_chunk - ac)
                for k4 in range(4):
                    rowbuf_ref[sg * 4 + k4:sg * 4 + k4 + 1, :] = jnp.sum(
                        accs[k4], axis=0, keepdims=True)
            dist_ref[pl.ds(pl.multiple_of(grp * 8, 8), 8),
                     pl.ds(nbase, _NSC)] = rowbuf_ref[...]
            return carry2

        lax.fori_loop(0, _TN // _NSC, nc_body, 0)
        return carry

    lax.fori_loop(0, _TA // 8, grp_body, 0)

    @pl.when(j == _NJ - 1)
    def _select_and_reduce():
        dvec = _GAMMA + jnp.sum(
            jnp.abs(ae1_ref[...] - ae2_ref[...]), axis=1, keepdims=True
        )

        def _di():
            return lax.bitcast_convert_type(dist_ref[...], jnp.int32)

        lo0 = jnp.min(_di(), axis=1, keepdims=True)
        hi0 = jnp.max(_di(), axis=1, keepdims=True)

        def bs_cond(lo_hi):
            lo, hi = lo_hi
            return jnp.any(lo < hi)

        def bs_body(lo_hi):
            lo, hi = lo_hi
            mid = lo + lax.shift_right_arithmetic(hi - lo, 1)
            cnt = jnp.sum((_di() <= mid).astype(jnp.int32),
                          axis=1, keepdims=True)
            ge = cnt >= _K
            return jnp.where(ge, lo, mid + 1), jnp.where(ge, mid, hi)

        tau_i, _ = lax.while_loop(bs_cond, bs_body, (lo0, hi0))
        tau_f = lax.bitcast_convert_type(tau_i, jnp.float32)

        d_f = dist_ref[...]
        below = _di() < tau_i
        c = jnp.sum(below.astype(jnp.int32), axis=1, keepdims=True)
        contrib = jnp.where(below, jnp.maximum(dvec - d_f, 0.0), 0.0)
        s = jnp.sum(contrib, axis=1, keepdims=True)
        total = s + (_K - c).astype(jnp.float32) * jnp.maximum(dvec - tau_f, 0.0)
        out_ref[...] = total


def _run_tc(ct, aet, ae1, ae2, interpret=False):
    return pl.pallas_call(
        _tc_body,
        grid=(2, _NI, _NJ),
        in_specs=[
            pl.BlockSpec((1, _D, _TN), lambda g, i, j: (g, 0, j)),
            pl.BlockSpec((1, _D, _TA), lambda g, i, j: (g, 0, i)),
            pl.BlockSpec((_TA, _D), lambda g, i, j: (i, 0)),
            pl.BlockSpec((_TA, _D), lambda g, i, j: (i, 0)),
        ],
        out_specs=pl.BlockSpec((_TA, 1), lambda g, i, j: (g * _NI + i, 0)),
        out_shape=jax.ShapeDtypeStruct((2 * _A, 1), jnp.float32),
        scratch_shapes=[pltpu.VMEM((_TA, _NPAD), jnp.float32),
                        pltpu.VMEM((8, _D, 1), jnp.float32),
                        pltpu.VMEM((8, _NSC), jnp.float32)],
        interpret=interpret,
    )(ct, aet, ae1, ae2)


def kernel(out1, out2, anchor1, anchor2):
    ae1, ae2 = _sc_gather(
        out1, out2, anchor1.astype(jnp.int32), anchor2.astype(jnp.int32)
    )
    pad = jnp.full((_NPAD - _N, _D), jnp.inf, jnp.float32)
    c2 = jnp.concatenate([out2, pad], axis=0)
    c1 = jnp.concatenate([out1, pad], axis=0)
    ct = jnp.stack([c2.T, c1.T])
    aet = jnp.stack([ae1.T, ae2.T])
    partial = _run_tc(ct, aet, ae1, ae2)
    return jnp.sum(partial) / (_A * _K)


# speedup vs baseline: 1.3091x; 1.0207x over previous
import functools

import jax
import jax.numpy as jnp
from jax import lax
from jax.experimental import pallas as pl
from jax.experimental.pallas import tpu as pltpu
from jax.experimental.pallas import tpu_sc as plsc

_K = 64
_GAMMA = 1.0
_N = 10000
_D = 256
_A = 1024
_TN = 2048
_NPAD = 10240
_TA = 128
_NI = _A // _TA
_NJ = _NPAD // _TN


def _sc_gather(t1, t2, i1, i2):
    info = plsc.get_sparse_core_info()
    nc, ns = info.num_cores, info.num_subcores
    nw = nc * ns
    bpw = _A // nw
    mesh = plsc.VectorSubcoreMesh(core_axis_name="c", subcore_axis_name="s")

    @functools.partial(
        pl.kernel,
        mesh=mesh,
        out_type=[
            jax.ShapeDtypeStruct((_A, _D), jnp.float32),
            jax.ShapeDtypeStruct((_A, _D), jnp.float32),
        ],
        scratch_types=[
            pltpu.VMEM((bpw,), jnp.int32),
            pltpu.VMEM((bpw, _D), jnp.float32),
            pltpu.SemaphoreType.DMA,
        ],
    )
    def gk(t1_hbm, t2_hbm, i1_hbm, i2_hbm, o1_hbm, o2_hbm, idx_v, rows_v, sem):
        wid = lax.axis_index("s") * nc + lax.axis_index("c")
        base = wid * bpw
        pltpu.sync_copy(i1_hbm.at[pl.ds(base, bpw)], idx_v)
        pltpu.async_copy(t1_hbm.at[idx_v], rows_v, sem).wait()
        pltpu.sync_copy(rows_v, o1_hbm.at[pl.ds(base, bpw)])
        pltpu.sync_copy(i2_hbm.at[pl.ds(base, bpw)], idx_v)
        pltpu.async_copy(t2_hbm.at[idx_v], rows_v, sem).wait()
        pltpu.sync_copy(rows_v, o2_hbm.at[pl.ds(base, bpw)])

    return gk(t1, t2, i1, i2)


_NSC = 512


def _tc_body(ct_ref, aet_ref, ae1_ref, ae2_ref, out_ref, dist_ref, acs_ref):
    j = pl.program_id(2)

    def grp_body(grp, carry):
        at = aet_ref[0]
        for kk in range(8):
            row_iota = lax.broadcasted_iota(jnp.int32, (_TA, 1), 0)
            oh = (row_iota == grp * 8 + kk).astype(jnp.float32)
            acs_ref[kk] = lax.dot_general(
                at, oh, (((1,), (0,)), ((), ())),
                preferred_element_type=jnp.float32,
            )

        def nc_body(nc, carry2):
            nbase = j * _TN + nc * _NSC
            accs = [jnp.zeros((8, _NSC), jnp.float32) for _ in range(8)]
            for dt in range(_D // 8):
                ct_chunk = ct_ref[0, dt * 8:(dt + 1) * 8,
                                  pl.ds(nc * _NSC, _NSC)]
                for kk in range(8):
                    ac = acs_ref[kk, dt * 8:(dt + 1) * 8, :]
                    accs[kk] = accs[kk] + jnp.abs(ct_chunk - ac)
            rows = [jnp.sum(a, axis=0, keepdims=True) for a in accs]
            blk = jnp.concatenate(rows, axis=0)
            dist_ref[pl.ds(pl.multiple_of(grp * 8, 8), 8),
                     pl.ds(nbase, _NSC)] = blk
            return carry2

        lax.fori_loop(0, _TN // _NSC, nc_body, 0)
        return carry

    lax.fori_loop(0, _TA // 8, grp_body, 0)

    @pl.when(j == _NJ - 1)
    def _select_and_reduce():
        dvec = _GAMMA + jnp.sum(
            jnp.abs(ae1_ref[...] - ae2_ref[...]), axis=1, keepdims=True
        )

        def _di():
            return lax.bitcast_convert_type(dist_ref[...], jnp.int32)

        lo0 = jnp.min(_di(), axis=1, keepdims=True)
        hi0 = jnp.max(_di(), axis=1, keepdims=True)

        def bs_cond(lo_hi):
            lo, hi = lo_hi
            return jnp.any(lo < hi)

        def bs_body(lo_hi):
            lo, hi = lo_hi
            mid = lo + lax.shift_right_arithmetic(hi - lo, 1)
            cnt = jnp.sum((_di() <= mid).astype(jnp.int32),
                          axis=1, keepdims=True)
            ge = cnt >= _K
            return jnp.where(ge, lo, mid + 1), jnp.where(ge, mid, hi)

        tau_i, _ = lax.while_loop(bs_cond, bs_body, (lo0, hi0))
        tau_f = lax.bitcast_convert_type(tau_i, jnp.float32)

        d_f = dist_ref[...]
        below = _di() < tau_i
        c = jnp.sum(below.astype(jnp.int32), axis=1, keepdims=True)
        contrib = jnp.where(below, jnp.maximum(dvec - d_f, 0.0), 0.0)
        s = jnp.sum(contrib, axis=1, keepdims=True)
        total = s + (_K - c).astype(jnp.float32) * jnp.maximum(dvec - tau_f, 0.0)
        out_ref[...] = total


def _run_tc(ct, aet, ae1, ae2, interpret=False):
    return pl.pallas_call(
        _tc_body,
        grid=(2, _NI, _NJ),
        in_specs=[
            pl.BlockSpec((1, _D, _TN), lambda g, i, j: (g, 0, j)),
            pl.BlockSpec((1, _D, _TA), lambda g, i, j: (g, 0, i)),
            pl.BlockSpec((_TA, _D), lambda g, i, j: (i, 0)),
            pl.BlockSpec((_TA, _D), lambda g, i, j: (i, 0)),
        ],
        out_specs=pl.BlockSpec((_TA, 1), lambda g, i, j: (g * _NI + i, 0)),
        out_shape=jax.ShapeDtypeStruct((2 * _A, 1), jnp.float32),
        scratch_shapes=[pltpu.VMEM((_TA, _NPAD), jnp.float32),
                        pltpu.VMEM((8, _D, 1), jnp.float32)],
        interpret=interpret,
    )(ct, aet, ae1, ae2)


def kernel(out1, out2, anchor1, anchor2):
    ae1, ae2 = _sc_gather(
        out1, out2, anchor1.astype(jnp.int32), anchor2.astype(jnp.int32)
    )
    pad = jnp.full((_NPAD - _N, _D), jnp.inf, jnp.float32)
    c2 = jnp.concatenate([out2, pad], axis=0)
    c1 = jnp.concatenate([out1, pad], axis=0)
    ct = jnp.stack([c2.T, c1.T])
    aet = jnp.stack([ae1.T, ae2.T])
    partial = _run_tc(ct, aet, ae1, ae2)
    return jnp.sum(partial) / (_A * _K)


# speedup vs baseline: 1.4424x; 1.1018x over previous
import functools

import jax
import jax.numpy as jnp
from jax import lax
from jax.experimental import pallas as pl
from jax.experimental.pallas import tpu as pltpu
from jax.experimental.pallas import tpu_sc as plsc

_K = 64
_GAMMA = 1.0
_N = 10000
_D = 256
_A = 1024
_TN = 2048
_NPAD = 10240
_TA = 128
_NI = _A // _TA
_NJ = _NPAD // _TN


def _sc_gather(t1, t2, i1, i2):
    info = plsc.get_sparse_core_info()
    nc, ns = info.num_cores, info.num_subcores
    nw = nc * ns
    bpw = _A // nw
    mesh = plsc.VectorSubcoreMesh(core_axis_name="c", subcore_axis_name="s")

    @functools.partial(
        pl.kernel,
        mesh=mesh,
        out_type=[
            jax.ShapeDtypeStruct((_A, _D), jnp.float32),
            jax.ShapeDtypeStruct((_A, _D), jnp.float32),
        ],
        scratch_types=[
            pltpu.VMEM((bpw,), jnp.int32),
            pltpu.VMEM((bpw, _D), jnp.float32),
            pltpu.SemaphoreType.DMA,
        ],
    )
    def gk(t1_hbm, t2_hbm, i1_hbm, i2_hbm, o1_hbm, o2_hbm, idx_v, rows_v, sem):
        wid = lax.axis_index("s") * nc + lax.axis_index("c")
        base = wid * bpw
        pltpu.sync_copy(i1_hbm.at[pl.ds(base, bpw)], idx_v)
        pltpu.async_copy(t1_hbm.at[idx_v], rows_v, sem).wait()
        pltpu.sync_copy(rows_v, o1_hbm.at[pl.ds(base, bpw)])
        pltpu.sync_copy(i2_hbm.at[pl.ds(base, bpw)], idx_v)
        pltpu.async_copy(t2_hbm.at[idx_v], rows_v, sem).wait()
        pltpu.sync_copy(rows_v, o2_hbm.at[pl.ds(base, bpw)])

    return gk(t1, t2, i1, i2)


_NSC = 512


def _tc_body(ct_ref, aet_ref, ae1_ref, ae2_ref, out_ref, dist_ref, acs_ref):
    j = pl.program_id(2)

    @pl.when(j == 0)
    def _extract_anchor_cols():
        at = aet_ref[0]

        def ex_body(grp, carry):
            for kk in range(8):
                row_iota = lax.broadcasted_iota(jnp.int32, (_TA, 1), 0)
                oh = (row_iota == grp * 8 + kk).astype(jnp.float32)
                col = lax.dot_general(
                    at, oh, (((1,), (0,)), ((), ())),
                    preferred_element_type=jnp.float32,
                )
                acs_ref[pl.ds(grp * 8 + kk, 1)] = col[None]
            return carry

        lax.fori_loop(0, _TA // 8, ex_body, 0)

    def grp_body(grp, carry):
        def nc_body(nc, carry2):
            nbase = j * _TN + nc * _NSC
            accs = [jnp.zeros((8, _NSC), jnp.float32) for _ in range(8)]
            for dt in range(_D // 8):
                ct_chunk = ct_ref[0, dt * 8:(dt + 1) * 8,
                                  pl.ds(nc * _NSC, _NSC)]
                for kk in range(8):
                    ac = acs_ref[grp * 8 + kk,
                                 dt * 8:(dt + 1) * 8, :]
                    accs[kk] = accs[kk] + jnp.abs(ct_chunk - ac)
            rows = [jnp.sum(a, axis=0, keepdims=True) for a in accs]
            blk = jnp.concatenate(rows, axis=0)
            dist_ref[pl.ds(pl.multiple_of(grp * 8, 8), 8),
                     pl.ds(nbase, _NSC)] = blk
            return carry2

        lax.fori_loop(0, _TN // _NSC, nc_body, 0)
        return carry

    lax.fori_loop(0, _TA // 8, grp_body, 0)

    @pl.when(j == _NJ - 1)
    def _select_and_reduce():
        dvec = _GAMMA + jnp.sum(
            jnp.abs(ae1_ref[...] - ae2_ref[...]), axis=1, keepdims=True
        )

        def _di():
            return lax.bitcast_convert_type(dist_ref[...], jnp.int32)

        lo0 = jnp.min(_di(), axis=1, keepdims=True)
        hi0 = jnp.max(_di(), axis=1, keepdims=True)

        def bs_cond(lo_hi):
            lo, hi = lo_hi
            return jnp.any(lo < hi)

        def bs_body(lo_hi):
            lo, hi = lo_hi
            mid = lo + lax.shift_right_arithmetic(hi - lo, 1)
            cnt = jnp.sum((_di() <= mid).astype(jnp.int32),
                          axis=1, keepdims=True)
            ge = cnt >= _K
            return jnp.where(ge, lo, mid + 1), jnp.where(ge, mid, hi)

        tau_i, _ = lax.while_loop(bs_cond, bs_body, (lo0, hi0))
        tau_f = lax.bitcast_convert_type(tau_i, jnp.float32)

        d_f = dist_ref[...]
        below = _di() < tau_i
        c = jnp.sum(below.astype(jnp.int32), axis=1, keepdims=True)
        contrib = jnp.where(below, jnp.maximum(dvec - d_f, 0.0), 0.0)
        s = jnp.sum(contrib, axis=1, keepdims=True)
        total = s + (_K - c).astype(jnp.float32) * jnp.maximum(dvec - tau_f, 0.0)
        out_ref[...] = total


def _run_tc(ct, aet, ae1, ae2, interpret=False):
    return pl.pallas_call(
        _tc_body,
        grid=(2, _NI, _NJ),
        in_specs=[
            pl.BlockSpec((1, _D, _TN), lambda g, i, j: (g, 0, j)),
            pl.BlockSpec((1, _D, _TA), lambda g, i, j: (g, 0, i)),
            pl.BlockSpec((_TA, _D), lambda g, i, j: (i, 0)),
            pl.BlockSpec((_TA, _D), lambda g, i, j: (i, 0)),
        ],
        out_specs=pl.BlockSpec((_TA, 1), lambda g, i, j: (g * _NI + i, 0)),
        out_shape=jax.ShapeDtypeStruct((2 * _A, 1), jnp.float32),
        scratch_shapes=[pltpu.VMEM((_TA, _NPAD), jnp.float32),
                        pltpu.VMEM((_TA, _D, 1), jnp.float32)],
        interpret=interpret,
    )(ct, aet, ae1, ae2)


def kernel(out1, out2, anchor1, anchor2):
    ae1, ae2 = _sc_gather(
        out1, out2, anchor1.astype(jnp.int32), anchor2.astype(jnp.int32)
    )
    pad = jnp.full((_NPAD - _N, _D), jnp.inf, jnp.float32)
    c2 = jnp.concatenate([out2, pad], axis=0)
    c1 = jnp.concatenate([out1, pad], axis=0)
    ct = jnp.stack([c2.T, c1.T])
    aet = jnp.stack([ae1.T, ae2.T])
    partial = _run_tc(ct, aet, ae1, ae2)
    return jnp.sum(partial) / (_A * _K)


# speedup vs baseline: 1.4440x; 1.0011x over previous
import functools

import jax
import jax.numpy as jnp
from jax import lax
from jax.experimental import pallas as pl
from jax.experimental.pallas import tpu as pltpu
from jax.experimental.pallas import tpu_sc as plsc

_K = 64
_GAMMA = 1.0
_N = 10000
_D = 256
_A = 1024
_TN = 5120
_NPAD = 10240
_TA = 128
_NI = _A // _TA
_NJ = _NPAD // _TN


def _sc_gather(t1, t2, i1, i2):
    info = plsc.get_sparse_core_info()
    nc, ns = info.num_cores, info.num_subcores
    nw = nc * ns
    bpw = _A // nw
    mesh = plsc.VectorSubcoreMesh(core_axis_name="c", subcore_axis_name="s")

    @functools.partial(
        pl.kernel,
        mesh=mesh,
        out_type=[
            jax.ShapeDtypeStruct((_A, _D), jnp.float32),
            jax.ShapeDtypeStruct((_A, _D), jnp.float32),
        ],
        scratch_types=[
            pltpu.VMEM((bpw,), jnp.int32),
            pltpu.VMEM((bpw, _D), jnp.float32),
            pltpu.SemaphoreType.DMA,
        ],
    )
    def gk(t1_hbm, t2_hbm, i1_hbm, i2_hbm, o1_hbm, o2_hbm, idx_v, rows_v, sem):
        wid = lax.axis_index("s") * nc + lax.axis_index("c")
        base = wid * bpw
        pltpu.sync_copy(i1_hbm.at[pl.ds(base, bpw)], idx_v)
        pltpu.async_copy(t1_hbm.at[idx_v], rows_v, sem).wait()
        pltpu.sync_copy(rows_v, o1_hbm.at[pl.ds(base, bpw)])
        pltpu.sync_copy(i2_hbm.at[pl.ds(base, bpw)], idx_v)
        pltpu.async_copy(t2_hbm.at[idx_v], rows_v, sem).wait()
        pltpu.sync_copy(rows_v, o2_hbm.at[pl.ds(base, bpw)])

    return gk(t1, t2, i1, i2)


_NSC = 512


def _tc_body(ct_ref, aet_ref, ae1_ref, ae2_ref, out_ref, dist_ref, acs_ref):
    j = pl.program_id(2)

    @pl.when(j == 0)
    def _extract_anchor_cols():
        at = aet_ref[0]

        def ex_body(grp, carry):
            for kk in range(8):
                row_iota = lax.broadcasted_iota(jnp.int32, (_TA, 1), 0)
                oh = (row_iota == grp * 8 + kk).astype(jnp.float32)
                col = lax.dot_general(
                    at, oh, (((1,), (0,)), ((), ())),
                    preferred_element_type=jnp.float32,
                )
                acs_ref[pl.ds(grp * 8 + kk, 1)] = col[None]
            return carry

        lax.fori_loop(0, _TA // 8, ex_body, 0)

    def grp_body(grp, carry):
        def nc_body(nc, carry2):
            nbase = j * _TN + nc * _NSC
            accs = [jnp.zeros((8, _NSC), jnp.float32) for _ in range(8)]
            for dt in range(_D // 8):
                ct_chunk = ct_ref[0, dt * 8:(dt + 1) * 8,
                                  pl.ds(nc * _NSC, _NSC)]
                for kk in range(8):
                    ac = acs_ref[grp * 8 + kk,
                                 dt * 8:(dt + 1) * 8, :]
                    accs[kk] = accs[kk] + jnp.abs(ct_chunk - ac)
            rows = [jnp.sum(a, axis=0, keepdims=True) for a in accs]
            blk = jnp.concatenate(rows, axis=0)
            dist_ref[pl.ds(pl.multiple_of(grp * 8, 8), 8),
                     pl.ds(nbase, _NSC)] = blk
            return carry2

        lax.fori_loop(0, _TN // _NSC, nc_body, 0)
        return carry

    lax.fori_loop(0, _TA // 8, grp_body, 0)

    @pl.when(j == _NJ - 1)
    def _select_and_reduce():
        dvec = _GAMMA + jnp.sum(
            jnp.abs(ae1_ref[...] - ae2_ref[...]), axis=1, keepdims=True
        )

        def _di():
            return lax.bitcast_convert_type(dist_ref[...], jnp.int32)

        lo0 = jnp.min(_di(), axis=1, keepdims=True)
        hi0 = jnp.max(_di(), axis=1, keepdims=True)

        def bs_cond(lo_hi):
            lo, hi = lo_hi
            return jnp.any(lo < hi)

        def bs_body(lo_hi):
            lo, hi = lo_hi
            mid = lo + lax.shift_right_arithmetic(hi - lo, 1)
            cnt = jnp.sum((_di() <= mid).astype(jnp.int32),
                          axis=1, keepdims=True)
            ge = cnt >= _K
            return jnp.where(ge, lo, mid + 1), jnp.where(ge, mid, hi)

        tau_i, _ = lax.while_loop(bs_cond, bs_body, (lo0, hi0))
        tau_f = lax.bitcast_convert_type(tau_i, jnp.float32)

        d_f = dist_ref[...]
        below = _di() < tau_i
        c = jnp.sum(below.astype(jnp.int32), axis=1, keepdims=True)
        contrib = jnp.where(below, jnp.maximum(dvec - d_f, 0.0), 0.0)
        s = jnp.sum(contrib, axis=1, keepdims=True)
        total = s + (_K - c).astype(jnp.float32) * jnp.maximum(dvec - tau_f, 0.0)
        out_ref[...] = total


def _run_tc(ct, aet, ae1, ae2, interpret=False):
    return pl.pallas_call(
        _tc_body,
        grid=(2, _NI, _NJ),
        in_specs=[
            pl.BlockSpec((1, _D, _TN), lambda g, i, j: (g, 0, j)),
            pl.BlockSpec((1, _D, _TA), lambda g, i, j: (g, 0, i)),
            pl.BlockSpec((_TA, _D), lambda g, i, j: (i, 0)),
            pl.BlockSpec((_TA, _D), lambda g, i, j: (i, 0)),
        ],
        out_specs=pl.BlockSpec((_TA, 1), lambda g, i, j: (g * _NI + i, 0)),
        out_shape=jax.ShapeDtypeStruct((2 * _A, 1), jnp.float32),
        scratch_shapes=[pltpu.VMEM((_TA, _NPAD), jnp.float32),
                        pltpu.VMEM((_TA, _D, 1), jnp.float32)],
        interpret=interpret,
    )(ct, aet, ae1, ae2)


def kernel(out1, out2, anchor1, anchor2):
    ae1, ae2 = _sc_gather(
        out1, out2, anchor1.astype(jnp.int32), anchor2.astype(jnp.int32)
    )
    pad = jnp.full((_NPAD - _N, _D), jnp.inf, jnp.float32)
    c2 = jnp.concatenate([out2, pad], axis=0)
    c1 = jnp.concatenate([out1, pad], axis=0)
    ct = jnp.stack([c2.T, c1.T])
    aet = jnp.stack([ae1.T, ae2.T])
    partial = _run_tc(ct, aet, ae1, ae2)
    return jnp.sum(partial) / (_A * _K)
